# baseline (speedup 1.0000x reference)
import functools

import jax
import jax.numpy as jnp
from jax import lax
from jax.experimental import pallas as pl
from jax.experimental.pallas import tpu as pltpu
from jax.experimental.pallas import tpu_sc as plsc

N_ROWS = 100000
D = 128
NUM_CORES = 2
NUM_SUBCORES = 16
CHUNK = 200
NCHUNKS = N_ROWS // CHUNK
CH_PER_SC = NCHUNKS // NUM_CORES
NROUNDS = (CH_PER_SC + NUM_SUBCORES - 1) // NUM_SUBCORES
SLAB = NUM_SUBCORES * CHUNK

_mesh = plsc.VectorSubcoreMesh(core_axis_name="c", subcore_axis_name="s")


@functools.partial(
    pl.kernel,
    mesh=_mesh,
    out_type=jax.ShapeDtypeStruct((N_ROWS, D), jnp.float32),
    scratch_types=[
        pltpu.VMEM((CHUNK,), jnp.int32),
        pltpu.VMEM((CHUNK,), jnp.int32),
        pltpu.VMEM((CHUNK, D), jnp.float32),
        pltpu.VMEM((CHUNK, D), jnp.float32),
        pltpu.VMEM_SHARED((2, SLAB, D), jnp.float32),
        pltpu.SemaphoreType.DMA,
        pltpu.SemaphoreType.DMA,
        pltpu.SemaphoreType.DMA,
        pltpu.SemaphoreType.DMA,
    ],
)
def _gather(table_hbm, idx_hbm, out_hbm, idx0, idx1, rows0, rows1, slab_v,
            gsem0, gsem1, wsem0, wsem1):
    cid = lax.axis_index("c")
    sid = lax.axis_index("s")
    idxs = (idx0, idx1)
    rows = (rows0, rows1)
    gsems = (gsem0, gsem1)
    wsems = (wsem0, wsem1)
    sc_base = cid * CH_PER_SC

    def round_chunks(r):
        return min(NUM_SUBCORES, CH_PER_SC - r * NUM_SUBCORES)

    def start_gather(r):
        b = r % 2
        c = sc_base + r * NUM_SUBCORES + sid

        @pl.when(r * NUM_SUBCORES + sid < CH_PER_SC)
        def _():
            pltpu.sync_copy(idx_hbm.at[pl.ds(c * CHUNK, CHUNK)], idxs[b])
            pltpu.async_copy(table_hbm.at[idxs[b]], rows[b], gsems[b])

    def stage_to_slab(r):
        b = r % 2

        @pl.when(r * NUM_SUBCORES + sid < CH_PER_SC)
        def _():
            pltpu.make_async_copy(table_hbm.at[idxs[b]], rows[b],
                                  gsems[b]).wait()
            pltpu.sync_copy(rows[b], slab_v.at[b, pl.ds(sid * CHUNK, CHUNK)])

    def write_descr(r):
        b = r % 2
        n = round_chunks(r) * CHUNK
        off = (sc_base + r * NUM_SUBCORES) * CHUNK
        return pltpu.make_async_copy(
            slab_v.at[b, pl.ds(0, n)], out_hbm.at[pl.ds(off, n)], wsems[b])

    start_gather(0)
    for r in range(NROUNDS):
        if r + 1 < NROUNDS:
            start_gather(r + 1)
        if r >= 2:
            @pl.when(sid == 0)
            def _():
                write_descr(r - 2).wait()
            plsc.subcore_barrier()
        stage_to_slab(r)
        plsc.subcore_barrier()

        @pl.when(sid == 0)
        def _():
            write_descr(r).start()
    @pl.when(sid == 0)
    def _():
        write_descr(NROUNDS - 2).wait()
        write_descr(NROUNDS - 1).wait()


def kernel(g, h, r, norm, W):
    idx = h.reshape(-1).astype(jnp.int32)
    return _gather(W, idx)


# speedup vs baseline: 1.0836x; 1.0836x over previous
import functools

import jax
import jax.numpy as jnp
from jax import lax
from jax.experimental import pallas as pl
from jax.experimental.pallas import tpu as pltpu
from jax.experimental.pallas import tpu_sc as plsc

N_ROWS = 100000
D = 128
NUM_CORES = 2
NUM_SUBCORES = 16
NW = NUM_CORES * NUM_SUBCORES
SPAN = 3200
CHUNK = 200
NFULL = SPAN // CHUNK
NLAST = (N_ROWS - (NW - 1) * SPAN) // CHUNK

_mesh = plsc.VectorSubcoreMesh(core_axis_name="c", subcore_axis_name="s")


@functools.partial(
    pl.kernel,
    mesh=_mesh,
    out_type=jax.ShapeDtypeStruct((N_ROWS, D), jnp.float32),
    scratch_types=[
        pltpu.VMEM((SPAN,), jnp.int32),
        pltpu.VMEM((2, CHUNK, D), jnp.float32),
        pltpu.SemaphoreType.DMA((2,)),
        pltpu.SemaphoreType.DMA((2,)),
    ],
)
def _gather(table_hbm, idx_hbm, out_hbm, idx_v, rows_v, gsem, wsem):
    wid = lax.axis_index("s") * NUM_CORES + lax.axis_index("c")
    base = wid * SPAN
    nch = jnp.where(wid == NW - 1, NLAST, NFULL)

    @pl.when(wid < NW - 1)
    def _():
        pltpu.sync_copy(idx_hbm.at[pl.ds(base, SPAN)], idx_v)

    @pl.when(wid == NW - 1)
    def _():
        pltpu.sync_copy(idx_hbm.at[pl.ds(base, NLAST * CHUNK)],
                        idx_v.at[pl.ds(0, NLAST * CHUNK)])

    def start_gather(j):
        b = lax.rem(j, 2)
        pltpu.async_copy(
            table_hbm.at[idx_v.at[pl.ds(j * CHUNK, CHUNK)]],
            rows_v.at[b], gsem.at[b])

    def wait_write(j):
        b = lax.rem(j, 2)
        pltpu.make_async_copy(
            rows_v.at[b], out_hbm.at[pl.ds(base + j * CHUNK, CHUNK)],
            wsem.at[b]).wait()

    start_gather(0)

    def step(j, carry):
        b = lax.rem(j, 2)

        @pl.when(j + 1 < nch)
        def _():
            @pl.when(j >= 1)
            def _():
                wait_write(j - 1)
            start_gather(j + 1)

        pltpu.make_async_copy(
            table_hbm.at[idx_v.at[pl.ds(j * CHUNK, CHUNK)]],
            rows_v.at[b], gsem.at[b]).wait()
        pltpu.async_copy(
            rows_v.at[b], out_hbm.at[pl.ds(base + j * CHUNK, CHUNK)],
            wsem.at[b])
        return carry

    lax.fori_loop(0, nch, step, 0)
    wait_write(nch - 2)
    wait_write(nch - 1)


def kernel(g, h, r, norm, W):
    idx = h.reshape(-1).astype(jnp.int32)
    return _gather(W, idx)


# speedup vs baseline: 1.0882x; 1.0043x over previous
import functools

import jax
import jax.numpy as jnp
from jax import lax
from jax.experimental import pallas as pl
from jax.experimental.pallas import tpu as pltpu
from jax.experimental.pallas import tpu_sc as plsc

N_ROWS = 100000
D = 128
NUM_CORES = 2
NUM_SUBCORES = 16
NW = NUM_CORES * NUM_SUBCORES
SPAN = 3200
CHUNK = 200
NFULL = SPAN // CHUNK
NLAST = (N_ROWS - (NW - 1) * SPAN) // CHUNK

_mesh = plsc.VectorSubcoreMesh(core_axis_name="c", subcore_axis_name="s")


@functools.partial(
    pl.kernel,
    mesh=_mesh,
    out_type=jax.ShapeDtypeStruct((N_ROWS, D), jnp.float32),
    scratch_types=[
        pltpu.VMEM((SPAN,), jnp.int32),
        pltpu.VMEM((2, CHUNK, D), jnp.float32),
        pltpu.SemaphoreType.DMA((2,)),
        pltpu.SemaphoreType.DMA((2,)),
    ],
)
def _gather(table_hbm, idx_hbm, out_hbm, idx_v, rows_v, gsem, wsem):
    wid = lax.axis_index("s") * NUM_CORES + lax.axis_index("c")
    base = wid * SPAN
    nch = jnp.where(wid == NW - 1, NLAST, NFULL)

    @pl.when(wid < NW - 1)
    def _():
        pltpu.sync_copy(idx_hbm.at[0, pl.ds(base, SPAN)], idx_v)

    @pl.when(wid == NW - 1)
    def _():
        pltpu.sync_copy(idx_hbm.at[0, pl.ds(base, 896)],
                        idx_v.at[pl.ds(0, 896)])

    def start_gather(j):
        b = lax.rem(j, 2)
        pltpu.async_copy(
            table_hbm.at[idx_v.at[pl.ds(j * CHUNK, CHUNK)]],
            rows_v.at[b], gsem.at[b])

    def wait_write(j):
        b = lax.rem(j, 2)
        pltpu.make_async_copy(
            rows_v.at[b], out_hbm.at[pl.ds(base + j * CHUNK, CHUNK)],
            wsem.at[b]).wait()

    start_gather(0)

    def step(j, carry):
        b = lax.rem(j, 2)

        @pl.when(j + 1 < nch)
        def _():
            @pl.when(j >= 1)
            def _():
                wait_write(j - 1)
            start_gather(j + 1)

        pltpu.make_async_copy(
            table_hbm.at[idx_v.at[pl.ds(j * CHUNK, CHUNK)]],
            rows_v.at[b], gsem.at[b]).wait()
        pltpu.async_copy(
            rows_v.at[b], out_hbm.at[pl.ds(base + j * CHUNK, CHUNK)],
            wsem.at[b])
        return carry

    lax.fori_loop(0, nch, step, 0)
    wait_write(nch - 2)
    wait_write(nch - 1)


def kernel(g, h, r, norm, W):
    idx = h.reshape(1, -1).astype(jnp.int32)
    return _gather(W, idx)


# speedup vs baseline: 1.0960x; 1.0071x over previous
import functools

import jax
import jax.numpy as jnp
from jax import lax
from jax.experimental import pallas as pl
from jax.experimental.pallas import tpu as pltpu
from jax.experimental.pallas import tpu_sc as plsc

N_ROWS = 100000
D = 128
NUM_CORES = 2
NUM_SUBCORES = 16
NW = NUM_CORES * NUM_SUBCORES
SPAN = 3200
CHUNK = 400
NFULL = SPAN // CHUNK
NLAST = (N_ROWS - (NW - 1) * SPAN) // CHUNK

_mesh = plsc.VectorSubcoreMesh(core_axis_name="c", subcore_axis_name="s")


@functools.partial(
    pl.kernel,
    mesh=_mesh,
    out_type=jax.ShapeDtypeStruct((N_ROWS, D), jnp.float32),
    scratch_types=[
        pltpu.VMEM((SPAN,), jnp.int32),
        pltpu.VMEM((2, CHUNK, D), jnp.float32),
        pltpu.SemaphoreType.DMA((2,)),
        pltpu.SemaphoreType.DMA((2,)),
    ],
)
def _gather(table_hbm, idx_hbm, out_hbm, idx_v, rows_v, gsem, wsem):
    wid = lax.axis_index("s") * NUM_CORES + lax.axis_index("c")
    base = wid * SPAN
    nch = jnp.where(wid == NW - 1, NLAST, NFULL)

    @pl.when(wid < NW - 1)
    def _():
        pltpu.sync_copy(idx_hbm.at[0, pl.ds(base, SPAN)], idx_v)

    @pl.when(wid == NW - 1)
    def _():
        pltpu.sync_copy(idx_hbm.at[0, pl.ds(base, 896)],
                        idx_v.at[pl.ds(0, 896)])

    def start_gather(j):
        b = lax.rem(j, 2)
        pltpu.async_copy(
            table_hbm.at[idx_v.at[pl.ds(j * CHUNK, CHUNK)]],
            rows_v.at[b], gsem.at[b])

    def wait_write(j):
        b = lax.rem(j, 2)
        pltpu.make_async_copy(
            rows_v.at[b], out_hbm.at[pl.ds(base + j * CHUNK, CHUNK)],
            wsem.at[b]).wait()

    start_gather(0)

    def step(j, carry):
        b = lax.rem(j, 2)

        @pl.when(j + 1 < nch)
        def _():
            @pl.when(j >= 1)
            def _():
                wait_write(j - 1)
            start_gather(j + 1)

        pltpu.make_async_copy(
            table_hbm.at[idx_v.at[pl.ds(j * CHUNK, CHUNK)]],
            rows_v.at[b], gsem.at[b]).wait()
        pltpu.async_copy(
            rows_v.at[b], out_hbm.at[pl.ds(base + j * CHUNK, CHUNK)],
            wsem.at[b])
        return carry

    lax.fori_loop(0, nch, step, 0)
    wait_write(nch - 2)
    wait_write(nch - 1)


def kernel(g, h, r, norm, W):
    idx = h.reshape(1, -1).astype(jnp.int32)
    return _gather(W, idx)
